# baseline (speedup 1.0000x reference)
import functools

import jax
import jax.numpy as jnp
from jax.experimental import pallas as pl
from jax.experimental.pallas import tpu as pltpu

N = 10000
E = 320000
D_IN = 128
H = 8
D_OUT = 64
HD = H * D_OUT


def _matmul_kernel(x_ref, w_ref, o_ref):
    o_ref[...] = jnp.dot(x_ref[...], w_ref[...],
                         preferred_element_type=jnp.float32)


def _feat(x, W):
    bn = 1000
    return pl.pallas_call(
        _matmul_kernel,
        grid=(N // bn,),
        in_specs=[pl.BlockSpec((bn, D_IN), lambda i: (i, 0)),
                  pl.BlockSpec((D_IN, HD), lambda i: (0, 0))],
        out_specs=pl.BlockSpec((bn, HD), lambda i: (i, 0)),
        out_shape=jax.ShapeDtypeStruct((N, HD), jnp.float32),
    )(x, W)


def _gat(x, W, attn_l, attn_r, bias, src, dst, w):
    feat = _feat(x, W).reshape(N, H, D_OUT)
    el = jnp.sum(feat * attn_l[None], axis=-1)
    er = jnp.sum(feat * attn_r[None], axis=-1)
    e = el[src] + er[dst]
    e = e * w[:, None]
    e = jnp.where(e > 0, e, 0.2 * e)
    m = jax.ops.segment_max(e, dst, num_segments=N)
    m = jnp.where(jnp.isfinite(m), m, 0.0)
    ex = jnp.exp(e - m[dst])
    denom = jax.ops.segment_sum(ex, dst, num_segments=N)
    a = ex / (denom[dst] + 1e-9)
    msg = feat[src] * a[:, :, None]
    rst = jax.ops.segment_sum(msg, dst, num_segments=N)
    rst = rst + bias.reshape(1, H, D_OUT)
    rst = jnp.where(rst > 0, rst, jnp.exp(jnp.minimum(rst, 0.0)) - 1.0)
    return rst.reshape(N, HD)


def kernel(x, edge_index_0, edge_weight_0, edge_index_1, edge_weight_1,
           W_0, attn_l_0, attn_r_0, bias_0,
           W_1, attn_l_1, attn_r_1, bias_1,
           sem_W1, sem_b1, sem_W2):
    z0 = _gat(x, W_0, attn_l_0, attn_r_0, bias_0,
              edge_index_0[0], edge_index_0[1], edge_weight_0)
    z1 = _gat(x, W_1, attn_l_1, attn_r_1, bias_1,
              edge_index_1[0], edge_index_1[1], edge_weight_1)
    z = jnp.stack([z0, z1], axis=1)
    w = jnp.tanh(z @ sem_W1 + sem_b1) @ sem_W2
    w = w.mean(axis=0)
    beta = jax.nn.softmax(w, axis=0)
    out = jnp.sum(beta[None] * z, axis=1)
    return out


# speedup vs baseline: 15.8056x; 15.8056x over previous
import functools

import jax
import jax.numpy as jnp
from jax import lax
from jax.experimental import pallas as pl
from jax.experimental.pallas import tpu as pltpu
from jax.experimental.pallas import tpu_sc as plsc

N = 10000
NP = 10240
E = 320000
D_IN = 128
H = 8
D_OUT = 64
HD = H * D_OUT
SEM_HID = 128

BN = 512
NBP = NP // BN
BN2 = 400
NB2 = N // BN2
NT = 16
EPT = E // NT
CC = 80
NCH = EPT // CC
NSL = NP // NT
DEN_SL = H * NP // NT

F32 = jnp.float32
I32 = jnp.int32


def _prep_body(x_ref, w_ref, al_ref, ar_ref, feat_ref, elr_ref):
    f = jnp.dot(x_ref[...], w_ref[0], preferred_element_type=F32)
    for k in range(4):
        feat_ref[k] = f[:, 128 * k:128 * (k + 1)]
    cols = []
    for h in range(H):
        fh = f[:, D_OUT * h:D_OUT * (h + 1)]
        cols.append(jnp.sum(fh * al_ref[0, h][None, :], axis=1, keepdims=True))
    for h in range(H):
        fh = f[:, D_OUT * h:D_OUT * (h + 1)]
        cols.append(jnp.sum(fh * ar_ref[0, h][None, :], axis=1, keepdims=True))
    cols.append(jnp.zeros((BN, 128 - 2 * H), F32))
    elr_ref[0] = jnp.concatenate(cols, axis=1)


def _prep(xp, Ws, als, ars):
    return pl.pallas_call(
        _prep_body,
        grid=(NBP, 2),
        in_specs=[
            pl.BlockSpec((BN, D_IN), lambda i, p: (i, 0)),
            pl.BlockSpec((1, D_IN, HD), lambda i, p: (p, 0, 0)),
            pl.BlockSpec((1, H, D_OUT), lambda i, p: (p, 0, 0)),
            pl.BlockSpec((1, H, D_OUT), lambda i, p: (p, 0, 0)),
        ],
        out_specs=[
            pl.BlockSpec((4, BN, 128), lambda i, p: (p, i, 0)),
            pl.BlockSpec((1, BN, 128), lambda i, p: (p, i, 0)),
        ],
        out_shape=[
            jax.ShapeDtypeStruct((8, NP, 128), F32),
            jax.ShapeDtypeStruct((2, NP, 128), F32),
        ],
    )(xp, Ws, als, ars)


_mesh = plsc.VectorSubcoreMesh(core_axis_name="c", subcore_axis_name="s")


@functools.partial(
    pl.kernel,
    out_type=[
        jax.ShapeDtypeStruct((2, 4, NP, 128), F32),
        jax.ShapeDtypeStruct((2 * H * E,), F32),
    ],
    mesh=_mesh,
    compiler_params=pltpu.CompilerParams(needs_layout_passes=False),
    scratch_types=[
        pltpu.VMEM((CC, 128), F32),
        pltpu.VMEM((CC, 128), F32),
        pltpu.VMEM((CC,), I32),
        pltpu.VMEM((1, CC), I32),
        pltpu.VMEM((CC,), I32),
        pltpu.VMEM((CC,), I32),
        pltpu.VMEM((CC,), F32),
        pltpu.VMEM((H * CC,), F32),
        pltpu.VMEM((H, CC), I32),
        pltpu.VMEM((2, CC), F32),
        pltpu.VMEM((CC,), I32),
        pltpu.VMEM((CC,), F32),
        pltpu.VMEM((CC,), F32),
        pltpu.VMEM((DEN_SL,), F32),
        pltpu.VMEM_SHARED((NP, 128), F32),
        pltpu.VMEM_SHARED((H * NP,), F32),
        pltpu.SemaphoreType.DMA,
        pltpu.SemaphoreType.DMA,
    ],
)
def _sc_edge(featflat, elrflat, srcs, dsts, wgts, rst, exh,
             elsb, erdb, srcb, dstb, srowb, drowb, wb, exb1, didx,
             exc, crow, ivals0, ivals1, dbuf,
             acc, den, gsem, ssem):
    cid = lax.axis_index("c")
    sid = lax.axis_index("s")
    tlo = sid * EPT
    iota = lax.iota(I32, 16)
    rot_idx = (iota + 8) % 16
    lmask8 = iota < 8
    iota_cc = iota * CC
    zeros16 = jnp.zeros((16,), F32)

    @pl.loop(0, DEN_SL // 16)
    def _(r):
        dbuf[pl.ds(r * 16, 16)] = zeros16

    pltpu.sync_copy(dbuf, den.at[pl.ds(sid * DEN_SL, DEN_SL)])
    plsc.subcore_barrier()

    noff = cid * NP

    @pl.loop(0, NCH)
    def _(c):
        base = tlo + c * CC
        ebase = cid * E + base
        pltpu.sync_copy(srcs.at[pl.ds(ebase, CC)], srcb)
        pltpu.sync_copy(dsts.at[pl.ds(ebase, CC)], dstb.at[0])
        pltpu.sync_copy(wgts.at[pl.ds(ebase, CC)], wb)
        for q in range(5):
            ds16 = pl.ds(q * 16, 16)
            srowb[ds16] = srcb[ds16] + noff
            drowb[ds16] = dstb[0, ds16] + noff
        pltpu.async_copy(elrflat.at[srowb], elsb, gsem).wait()
        pltpu.async_copy(elrflat.at[drowb], erdb, gsem).wait()

        @pl.loop(0, 5)
        def _(q):
            q16 = q * 16
            wv = wb[pl.ds(q16, 16)]
            dv = dstb[0, pl.ds(q16, 16)]
            for h in range(H):
                didx[h, pl.ds(q16, 16)] = dv + h * NP
            for j in range(16):
                er_ = q16 + j
                ev = elsb[er_, pl.ds(0, 16)]
                rv = erdb[er_, pl.ds(0, 16)]
                s = (ev + jnp.take(rv, rot_idx)) * wv[j]
                s = jnp.where(s > 0, s, 0.2 * s)
                exv = jnp.exp(s)
                plsc.store_scatter(exb1, [iota_cc + er_], exv, mask=lmask8)

        cb = ((cid * NT + sid) * NCH + c) * (H * CC)
        pltpu.sync_copy(exb1, exh.at[pl.ds(cb, H * CC)])
        waits = [pltpu.async_copy(exb1.at[pl.ds(h * CC, CC)],
                                  den.at[didx.at[h]], ssem, add=True)
                 for h in range(H)]
        for wd in waits:
            wd.wait()

    plsc.subcore_barrier()

    d0 = sid * DEN_SL
    pltpu.sync_copy(den.at[pl.ds(d0, DEN_SL)], dbuf)

    @pl.loop(0, DEN_SL // 16)
    def _(r):
        ds16 = pl.ds(r * 16, 16)
        dbuf[ds16] = 1.0 / (dbuf[ds16] + 1e-9)

    pltpu.sync_copy(dbuf, den.at[pl.ds(d0, DEN_SL)])
    plsc.subcore_barrier()

    @pl.loop(0, CC)
    def _(r):
        for q in range(8):
            erdb[r, pl.ds(q * 16, 16)] = zeros16

    @pl.loop(0, 4)
    def _(g):
        h0 = 2 * g
        for k in range(NSL // CC):
            pltpu.sync_copy(erdb, acc.at[pl.ds(sid * NSL + k * CC, CC), :])
        plsc.subcore_barrier()
        goff = (cid * 4 + g) * NP

        @pl.loop(0, NCH)
        def _(c):
            base = tlo + c * CC
            ebase = cid * E + base
            cb = ((cid * NT + sid) * NCH + c) * (H * CC)
            pltpu.sync_copy(srcs.at[pl.ds(ebase, CC)], srcb)
            pltpu.sync_copy(dsts.at[pl.ds(ebase, CC)], dstb.at[0])
            pltpu.sync_copy(exh.at[pl.ds(cb + h0 * CC, CC)], exc.at[0])
            pltpu.sync_copy(exh.at[pl.ds(cb + (h0 + 1) * CC, CC)], exc.at[1])
            for q in range(5):
                ds16 = pl.ds(q * 16, 16)
                crow[ds16] = srcb[ds16] + goff
                dv = dstb[0, ds16]
                didx[0, ds16] = dv + h0 * NP
                didx[1, ds16] = dv + (h0 + 1) * NP
            cg = pltpu.async_copy(featflat.at[crow], elsb, gsem)
            g0 = pltpu.async_copy(den.at[didx.at[0]], ivals0, ssem)
            g1 = pltpu.async_copy(den.at[didx.at[1]], ivals1, ssem)
            cg.wait()
            g0.wait()
            g1.wait()

            @pl.loop(0, 5)
            def _(q):
                q16 = q * 16
                ds16 = pl.ds(q16, 16)
                a0 = exc[0, ds16] * ivals0[ds16]
                a1 = exc[1, ds16] * ivals1[ds16]
                for j in range(16):
                    er_ = q16 + j
                    s0 = a0[j]
                    s1 = a1[j]
                    for k in range(4):
                        dsk = pl.ds(k * 16, 16)
                        elsb[er_, dsk] = elsb[er_, dsk] * s0
                    for k in range(4, 8):
                        dsk = pl.ds(k * 16, 16)
                        elsb[er_, dsk] = elsb[er_, dsk] * s1

            pltpu.sync_copy(elsb, acc.at[dstb.at[0]], add=True)

        plsc.subcore_barrier()
        r0 = sid * NSL
        pltpu.sync_copy(acc.at[pl.ds(r0, NSL), :],
                        rst.at[cid, g, pl.ds(r0, NSL), :])
        plsc.subcore_barrier()


def _elu_z(rst_ref, bs_ref, p):
    z = jnp.concatenate([rst_ref[p, g] for g in range(4)], axis=1)
    z = z + bs_ref[p][None, :]
    return jnp.where(z > 0, z, jnp.exp(jnp.minimum(z, 0.0)) - 1.0)


def _sem_partial_body(rst_ref, bs_ref, w1_ref, b1_ref, w2_ref, o_ref):
    outs = []
    for p in range(2):
        z = _elu_z(rst_ref, bs_ref, p)
        t = jnp.tanh(jnp.dot(z, w1_ref[...], preferred_element_type=F32)
                     + b1_ref[0][None, :])
        s = jnp.sum(t * w2_ref[:, 0][None, :])
        outs.append(s.reshape(1, 1, 1))
    o_ref[...] = jnp.concatenate(outs, axis=2)


def _sem_partial(rst, bs, sW1, sb1, sW2):
    return pl.pallas_call(
        _sem_partial_body,
        grid=(NB2,),
        in_specs=[
            pl.BlockSpec((2, 4, BN2, 128), lambda i: (0, 0, i, 0)),
            pl.BlockSpec((2, HD), lambda i: (0, 0)),
            pl.BlockSpec((HD, SEM_HID), lambda i: (0, 0)),
            pl.BlockSpec((1, SEM_HID), lambda i: (0, 0)),
            pl.BlockSpec((SEM_HID, 1), lambda i: (0, 0)),
        ],
        out_specs=pl.BlockSpec((1, 1, 2), lambda i: (i, 0, 0)),
        out_shape=jax.ShapeDtypeStruct((NB2, 1, 2), F32),
    )(rst, bs, sW1, sb1, sW2)


def _combine_body(rst_ref, bs_ref, p_ref, o_ref):
    s = jnp.sum(p_ref[...], axis=0) / N
    s = s - jnp.max(s)
    bexp = jnp.exp(s)
    beta = bexp / jnp.sum(bexp)
    z0 = _elu_z(rst_ref, bs_ref, 0)
    z1 = _elu_z(rst_ref, bs_ref, 1)
    o_ref[...] = z0 * beta[0] + z1 * beta[1]


def _combine(rst, bs, partials):
    return pl.pallas_call(
        _combine_body,
        grid=(NB2,),
        in_specs=[
            pl.BlockSpec((2, 4, BN2, 128), lambda i: (0, 0, i, 0)),
            pl.BlockSpec((2, HD), lambda i: (0, 0)),
            pl.BlockSpec((NB2, 2), lambda i: (0, 0)),
        ],
        out_specs=pl.BlockSpec((BN2, HD), lambda i: (i, 0)),
        out_shape=jax.ShapeDtypeStruct((N, HD), F32),
    )(rst, bs, partials)


def kernel(x, edge_index_0, edge_weight_0, edge_index_1, edge_weight_1,
           W_0, attn_l_0, attn_r_0, bias_0,
           W_1, attn_l_1, attn_r_1, bias_1,
           sem_W1, sem_b1, sem_W2):
    Ws = jnp.stack([W_0, W_1])
    als = jnp.stack([attn_l_0, attn_l_1])
    ars = jnp.stack([attn_r_0, attn_r_1])
    xp = jnp.pad(x, ((0, NP - N), (0, 0)))
    featg, elr = _prep(xp, Ws, als, ars)
    featflat = featg.reshape(8 * NP, 128)
    elrflat = elr.reshape(2 * NP, 128)
    srcs = jnp.concatenate([edge_index_0[0], edge_index_1[0]]).astype(I32)
    dsts = jnp.concatenate([edge_index_0[1], edge_index_1[1]]).astype(I32)
    wgts = jnp.concatenate([edge_weight_0, edge_weight_1])
    rst, _ex = _sc_edge(featflat, elrflat, srcs, dsts, wgts)
    bs = jnp.stack([bias_0, bias_1])
    partials = _sem_partial(rst, bs, sem_W1, sem_b1.reshape(1, SEM_HID), sem_W2)
    return _combine(rst, bs, partials.reshape(NB2, 2))


# speedup vs baseline: 29.5434x; 1.8692x over previous
import functools

import jax
import jax.numpy as jnp
from jax import lax
from jax.experimental import pallas as pl
from jax.experimental.pallas import tpu as pltpu
from jax.experimental.pallas import tpu_sc as plsc

N = 10000
NP = 10240
E = 320000
D_IN = 128
H = 8
D_OUT = 64
HD = H * D_OUT
SEM_HID = 128

BN = 512
NBP = NP // BN
BN2 = 400
NB2 = N // BN2
NT = 16
CC = 128
NCH = E // CC
NFULL = NCH // NT
NSL = NP // NT
DEN_SL = H * NP // NT

F32 = jnp.float32
I32 = jnp.int32


def _prep_body(x_ref, w_ref, al_ref, ar_ref, feat_ref, elr_ref):
    f = jnp.dot(x_ref[...], w_ref[0], preferred_element_type=F32)
    for k in range(4):
        feat_ref[k] = f[:, 128 * k:128 * (k + 1)]
    cols = []
    for h in range(H):
        fh = f[:, D_OUT * h:D_OUT * (h + 1)]
        cols.append(jnp.sum(fh * al_ref[0, h][None, :], axis=1, keepdims=True))
    for h in range(H):
        fh = f[:, D_OUT * h:D_OUT * (h + 1)]
        cols.append(jnp.sum(fh * ar_ref[0, h][None, :], axis=1, keepdims=True))
    cols.append(jnp.zeros((BN, 128 - 2 * H), F32))
    elr_ref[0] = jnp.concatenate(cols, axis=1)


def _prep(xp, Ws, als, ars):
    return pl.pallas_call(
        _prep_body,
        grid=(NBP, 2),
        in_specs=[
            pl.BlockSpec((BN, D_IN), lambda i, p: (i, 0)),
            pl.BlockSpec((1, D_IN, HD), lambda i, p: (p, 0, 0)),
            pl.BlockSpec((1, H, D_OUT), lambda i, p: (p, 0, 0)),
            pl.BlockSpec((1, H, D_OUT), lambda i, p: (p, 0, 0)),
        ],
        out_specs=[
            pl.BlockSpec((4, BN, 128), lambda i, p: (p, i, 0)),
            pl.BlockSpec((1, BN, 128), lambda i, p: (p, i, 0)),
        ],
        out_shape=[
            jax.ShapeDtypeStruct((8, NP, 128), F32),
            jax.ShapeDtypeStruct((2, NP, 128), F32),
        ],
    )(xp, Ws, als, ars)


_mesh = plsc.VectorSubcoreMesh(core_axis_name="c", subcore_axis_name="s")


@functools.partial(
    pl.kernel,
    out_type=[
        jax.ShapeDtypeStruct((2, 4, NP, 128), F32),
        jax.ShapeDtypeStruct((2 * H * E,), F32),
    ],
    mesh=_mesh,
    compiler_params=pltpu.CompilerParams(needs_layout_passes=False),
    scratch_types=[
        pltpu.VMEM((CC, 128), F32),
        pltpu.VMEM((CC, 128), F32),
        pltpu.VMEM((2, CC), I32),
        pltpu.VMEM((2, CC), I32),
        pltpu.VMEM((CC,), F32),
        pltpu.VMEM((H * CC,), F32),
        pltpu.VMEM((H, CC), I32),
        pltpu.VMEM((2, 2 * CC), F32),
        pltpu.VMEM((2, CC), I32),
        pltpu.VMEM((2, CC), F32),
        pltpu.VMEM((2, CC), F32),
        pltpu.VMEM((CC,), I32),
        pltpu.VMEM((CC,), I32),
        pltpu.VMEM((DEN_SL,), F32),
        pltpu.VMEM_SHARED((NP, 128), F32),
        pltpu.VMEM_SHARED((H * NP,), F32),
        pltpu.SemaphoreType.DMA,
        pltpu.SemaphoreType.DMA,
        pltpu.SemaphoreType.DMA,
        pltpu.SemaphoreType.DMA,
        pltpu.SemaphoreType.DMA,
        pltpu.SemaphoreType.DMA,
        pltpu.SemaphoreType.DMA,
    ],
)
def _sc_edge(featflat, elrflat, srcs, dsts, wgts, zrows, rst, exh,
             fb0, fb1, src2, dst2, wb, exb1, didx, exc2, crow2, iv0, iv1,
             srowb, drowb, dbuf, acc, den,
             st0, st1, g0s, g1s, i0s, i1s, ssem):
    cid = lax.axis_index("c")
    sid = lax.axis_index("s")
    iota = lax.iota(I32, 16)
    rot_idx = (iota + 8) % 16
    lmask8 = iota < 8
    iota_cc = iota * CC
    zeros16 = jnp.zeros((16,), F32)
    NG = CC // 16
    NTAIL = NCH - NFULL * NT
    fbs = (fb0, fb1)
    gsems = (g0s, g1s)
    isems = (i0s, i1s)
    stsems = (st0, st1)
    nch_t = jnp.where(sid < NTAIL, NFULL + 1, NFULL)

    @pl.loop(0, DEN_SL // 16)
    def _(r):
        dbuf[pl.ds(r * 16, 16)] = zeros16

    pltpu.sync_copy(dbuf, den.at[pl.ds(sid * DEN_SL, DEN_SL)])
    plsc.subcore_barrier()

    def do_chunk_a(k):
        c = sid + NT * k
        ebase = cid * E + c * CC
        d1 = pltpu.async_copy(srcs.at[pl.ds(ebase, CC)], src2.at[0], st0)
        d2 = pltpu.async_copy(dsts.at[pl.ds(ebase, CC)], dst2.at[0], st0)
        d3 = pltpu.async_copy(wgts.at[pl.ds(ebase, CC)], wb, st0)
        d1.wait()
        d2.wait()
        d3.wait()
        for q in range(NG):
            ds16 = pl.ds(q * 16, 16)
            srowb[ds16] = src2[0, ds16] + cid * NP
            drowb[ds16] = dst2[0, ds16] + cid * NP
        e1 = pltpu.async_copy(elrflat.at[srowb], fb0, g0s)
        e2 = pltpu.async_copy(elrflat.at[drowb], fb1, g1s)
        e1.wait()
        e2.wait()

        @pl.loop(0, NG)
        def _(q):
            q16 = q * 16
            wv = wb[pl.ds(q16, 16)]
            dv = dst2[0, pl.ds(q16, 16)]
            for h in range(H):
                didx[h, pl.ds(q16, 16)] = dv + h * NP
            for j in range(16):
                er_ = q16 + j
                ev = fb0[er_, pl.ds(0, 16)]
                rv = fb1[er_, pl.ds(0, 16)]
                s = (ev + jnp.take(rv, rot_idx)) * wv[j]
                s = jnp.where(s > 0, s, 0.2 * s)
                plsc.store_scatter(exb1, [iota_cc + er_], jnp.exp(s),
                                   mask=lmask8)

        cb = (cid * NCH + c) * (H * CC)
        wx = pltpu.async_copy(exb1, exh.at[pl.ds(cb, H * CC)], st1)
        waits = [pltpu.async_copy(exb1.at[pl.ds(h * CC, CC)],
                                  den.at[didx.at[h]], ssem, add=True)
                 for h in range(H)]
        wx.wait()
        for wd in waits:
            wd.wait()

    @pl.loop(0, NFULL)
    def _(k):
        do_chunk_a(k)

    @pl.when(sid < NTAIL)
    def _():
        do_chunk_a(NFULL)

    plsc.subcore_barrier()

    d0 = sid * DEN_SL
    pltpu.sync_copy(den.at[pl.ds(d0, DEN_SL)], dbuf)

    @pl.loop(0, DEN_SL // 16)
    def _(r):
        ds16 = pl.ds(r * 16, 16)
        dbuf[ds16] = 1.0 / (dbuf[ds16] + 1e-9)

    pltpu.sync_copy(dbuf, den.at[pl.ds(d0, DEN_SL)])
    plsc.subcore_barrier()

    @pl.loop(0, 4)
    def _(g):
        h0 = 2 * g
        pltpu.sync_copy(zrows, acc.at[pl.ds(sid * NSL, NSL), :])
        plsc.subcore_barrier()
        goff = (cid * 4 + g) * NP

        def stage_c(k, b):
            c = sid + NT * k
            ebase = cid * E + c * CC
            cb = (cid * NCH + c) * (H * CC)
            return [
                pltpu.async_copy(srcs.at[pl.ds(ebase, CC)], src2.at[b],
                                 stsems[b]),
                pltpu.async_copy(dsts.at[pl.ds(ebase, CC)], dst2.at[b],
                                 stsems[b]),
                pltpu.async_copy(exh.at[pl.ds(cb + h0 * CC, 2 * CC)],
                                 exc2.at[b], stsems[b]),
            ]

        def wait_stage_c(k, b):
            c = sid + NT * k
            ebase = cid * E + c * CC
            cb = (cid * NCH + c) * (H * CC)
            pltpu.make_async_copy(srcs.at[pl.ds(ebase, CC)], src2.at[b],
                                  stsems[b]).wait()
            pltpu.make_async_copy(dsts.at[pl.ds(ebase, CC)], dst2.at[b],
                                  stsems[b]).wait()
            pltpu.make_async_copy(exh.at[pl.ds(cb + h0 * CC, 2 * CC)],
                                  exc2.at[b], stsems[b]).wait()

        def fire_gathers_c(b):
            for q in range(NG):
                ds16 = pl.ds(q * 16, 16)
                dv = dst2[b, ds16]
                crow2[b, ds16] = src2[b, ds16] + goff
                didx[2 * b, ds16] = dv + h0 * NP
                didx[2 * b + 1, ds16] = dv + (h0 + 1) * NP
            pltpu.async_copy(featflat.at[crow2.at[b]], fbs[b], gsems[b])
            pltpu.async_copy(den.at[didx.at[2 * b]], iv0.at[b], isems[b])
            pltpu.async_copy(den.at[didx.at[2 * b + 1]], iv1.at[b], isems[b])

        def consume_c(b):
            fb = fbs[b]
            pltpu.make_async_copy(featflat.at[crow2.at[b]], fb,
                                  gsems[b]).wait()
            pltpu.make_async_copy(den.at[didx.at[2 * b]], iv0.at[b],
                                  isems[b]).wait()
            pltpu.make_async_copy(den.at[didx.at[2 * b + 1]], iv1.at[b],
                                  isems[b]).wait()

            @pl.loop(0, NG)
            def _(q):
                q16 = q * 16
                ds16 = pl.ds(q16, 16)
                a0 = exc2[b, ds16] * iv0[b, ds16]
                a1 = exc2[b, pl.ds(CC + q16, 16)] * iv1[b, ds16]
                for j in range(16):
                    er_ = q16 + j
                    s0 = a0[j]
                    s1 = a1[j]
                    for kk in range(4):
                        dsk = pl.ds(kk * 16, 16)
                        fb[er_, dsk] = fb[er_, dsk] * s0
                    for kk in range(4, 8):
                        dsk = pl.ds(kk * 16, 16)
                        fb[er_, dsk] = fb[er_, dsk] * s1

            pltpu.sync_copy(fb, acc.at[dst2.at[b]], add=True)

        for d in stage_c(0, 0):
            d.wait()
        fire_gathers_c(0)

        @pl.loop(0, NFULL // 2)
        def _(m):
            k0 = 2 * m
            stage_c(k0 + 1, 1)
            consume_c(0)
            wait_stage_c(k0 + 1, 1)
            fire_gathers_c(1)
            knext = k0 + 2

            @pl.when(knext < nch_t)
            def _():
                stage_c(knext, 0)
            consume_c(1)

            @pl.when(knext < nch_t)
            def _():
                wait_stage_c(knext, 0)
                fire_gathers_c(0)

        @pl.when(sid < NTAIL)
        def _():
            consume_c(0)

        plsc.subcore_barrier()
        r0 = sid * NSL
        pltpu.sync_copy(acc.at[pl.ds(r0, NSL), :],
                        rst.at[cid, g, pl.ds(r0, NSL), :])
        plsc.subcore_barrier()


def _elu_z(rst_ref, bs_ref, p):
    z = jnp.concatenate([rst_ref[p, g] for g in range(4)], axis=1)
    z = z + bs_ref[p][None, :]
    return jnp.where(z > 0, z, jnp.exp(jnp.minimum(z, 0.0)) - 1.0)


def _sem_partial_body(rst_ref, bs_ref, w1_ref, b1_ref, w2_ref, o_ref):
    outs = []
    for p in range(2):
        z = _elu_z(rst_ref, bs_ref, p)
        t = jnp.tanh(jnp.dot(z, w1_ref[...], preferred_element_type=F32)
                     + b1_ref[0][None, :])
        s = jnp.sum(t * w2_ref[:, 0][None, :])
        outs.append(s.reshape(1, 1, 1))
    o_ref[...] = jnp.concatenate(outs, axis=2)


def _sem_partial(rst, bs, sW1, sb1, sW2):
    return pl.pallas_call(
        _sem_partial_body,
        grid=(NB2,),
        in_specs=[
            pl.BlockSpec((2, 4, BN2, 128), lambda i: (0, 0, i, 0)),
            pl.BlockSpec((2, HD), lambda i: (0, 0)),
            pl.BlockSpec((HD, SEM_HID), lambda i: (0, 0)),
            pl.BlockSpec((1, SEM_HID), lambda i: (0, 0)),
            pl.BlockSpec((SEM_HID, 1), lambda i: (0, 0)),
        ],
        out_specs=pl.BlockSpec((1, 1, 2), lambda i: (i, 0, 0)),
        out_shape=jax.ShapeDtypeStruct((NB2, 1, 2), F32),
    )(rst, bs, sW1, sb1, sW2)


def _combine_body(rst_ref, bs_ref, p_ref, o_ref):
    s = jnp.sum(p_ref[...], axis=0) / N
    s = s - jnp.max(s)
    bexp = jnp.exp(s)
    beta = bexp / jnp.sum(bexp)
    z0 = _elu_z(rst_ref, bs_ref, 0)
    z1 = _elu_z(rst_ref, bs_ref, 1)
    o_ref[...] = z0 * beta[0] + z1 * beta[1]


def _combine(rst, bs, partials):
    return pl.pallas_call(
        _combine_body,
        grid=(NB2,),
        in_specs=[
            pl.BlockSpec((2, 4, BN2, 128), lambda i: (0, 0, i, 0)),
            pl.BlockSpec((2, HD), lambda i: (0, 0)),
            pl.BlockSpec((NB2, 2), lambda i: (0, 0)),
        ],
        out_specs=pl.BlockSpec((BN2, HD), lambda i: (i, 0)),
        out_shape=jax.ShapeDtypeStruct((N, HD), F32),
    )(rst, bs, partials)


def kernel(x, edge_index_0, edge_weight_0, edge_index_1, edge_weight_1,
           W_0, attn_l_0, attn_r_0, bias_0,
           W_1, attn_l_1, attn_r_1, bias_1,
           sem_W1, sem_b1, sem_W2):
    Ws = jnp.stack([W_0, W_1])
    als = jnp.stack([attn_l_0, attn_l_1])
    ars = jnp.stack([attn_r_0, attn_r_1])
    xp = jnp.pad(x, ((0, NP - N), (0, 0)))
    featg, elr = _prep(xp, Ws, als, ars)
    featflat = featg.reshape(8 * NP, 128)
    elrflat = elr.reshape(2 * NP, 128)
    srcs = jnp.concatenate([edge_index_0[0], edge_index_1[0]]).astype(I32)
    dsts = jnp.concatenate([edge_index_0[1], edge_index_1[1]]).astype(I32)
    wgts = jnp.concatenate([edge_weight_0, edge_weight_1])
    zrows = jnp.zeros((NSL, 128), F32)
    rst, _ex = _sc_edge(featflat, elrflat, srcs, dsts, wgts, zrows)
    bs = jnp.stack([bias_0, bias_1])
    partials = _sem_partial(rst, bs, sem_W1, sem_b1.reshape(1, SEM_HID), sem_W2)
    return _combine(rst, bs, partials.reshape(NB2, 2))


# speedup vs baseline: 29.5578x; 1.0005x over previous
import functools

import jax
import jax.numpy as jnp
from jax import lax
from jax.experimental import pallas as pl
from jax.experimental.pallas import tpu as pltpu
from jax.experimental.pallas import tpu_sc as plsc

N = 10000
NP = 10240
E = 320000
D_IN = 128
H = 8
D_OUT = 64
HD = H * D_OUT
SEM_HID = 128

BN = 512
NBP = NP // BN
BN2 = 400
NB2 = N // BN2
NT = 16
CC = 128
NCH = E // CC
NFULL = NCH // NT
NSL = NP // NT
DEN_SL = H * NP // NT

F32 = jnp.float32
I32 = jnp.int32


def _prep_body(x_ref, w_ref, al_ref, ar_ref, feat_ref, elr_ref):
    f = jnp.dot(x_ref[...], w_ref[0], preferred_element_type=F32)
    for k in range(4):
        feat_ref[k] = f[:, 128 * k:128 * (k + 1)]
    cols = []
    for h in range(H):
        fh = f[:, D_OUT * h:D_OUT * (h + 1)]
        cols.append(jnp.sum(fh * al_ref[0, h][None, :], axis=1, keepdims=True))
    for h in range(H):
        fh = f[:, D_OUT * h:D_OUT * (h + 1)]
        cols.append(jnp.sum(fh * ar_ref[0, h][None, :], axis=1, keepdims=True))
    cols.append(jnp.zeros((BN, 128 - 2 * H), F32))
    elr_ref[0] = jnp.concatenate(cols, axis=1)


def _prep(xp, Ws, als, ars):
    return pl.pallas_call(
        _prep_body,
        grid=(NBP, 2),
        in_specs=[
            pl.BlockSpec((BN, D_IN), lambda i, p: (i, 0)),
            pl.BlockSpec((1, D_IN, HD), lambda i, p: (p, 0, 0)),
            pl.BlockSpec((1, H, D_OUT), lambda i, p: (p, 0, 0)),
            pl.BlockSpec((1, H, D_OUT), lambda i, p: (p, 0, 0)),
        ],
        out_specs=[
            pl.BlockSpec((4, BN, 128), lambda i, p: (p, i, 0)),
            pl.BlockSpec((1, BN, 128), lambda i, p: (p, i, 0)),
        ],
        out_shape=[
            jax.ShapeDtypeStruct((8, NP, 128), F32),
            jax.ShapeDtypeStruct((2, NP, 128), F32),
        ],
    )(xp, Ws, als, ars)


_mesh = plsc.VectorSubcoreMesh(core_axis_name="c", subcore_axis_name="s")


@functools.partial(
    pl.kernel,
    out_type=[
        jax.ShapeDtypeStruct((2, 4, NP, 128), F32),
        jax.ShapeDtypeStruct((2 * H * E,), F32),
    ],
    mesh=_mesh,
    compiler_params=pltpu.CompilerParams(needs_layout_passes=False),
    scratch_types=[
        pltpu.VMEM((CC, 128), F32),
        pltpu.VMEM((CC, 128), F32),
        pltpu.VMEM((2, CC), I32),
        pltpu.VMEM((2, CC), I32),
        pltpu.VMEM((CC,), F32),
        pltpu.VMEM((H * CC,), F32),
        pltpu.VMEM((H, CC), I32),
        pltpu.VMEM((2, 2 * CC), F32),
        pltpu.VMEM((2, CC), I32),
        pltpu.VMEM((2, CC), F32),
        pltpu.VMEM((2, CC), F32),
        pltpu.VMEM((CC,), I32),
        pltpu.VMEM((CC,), I32),
        pltpu.VMEM((DEN_SL,), F32),
        pltpu.VMEM_SHARED((NP, 128), F32),
        pltpu.VMEM_SHARED((H * NP,), F32),
        pltpu.SemaphoreType.DMA,
        pltpu.SemaphoreType.DMA,
        pltpu.SemaphoreType.DMA,
        pltpu.SemaphoreType.DMA,
        pltpu.SemaphoreType.DMA,
        pltpu.SemaphoreType.DMA,
        pltpu.SemaphoreType.DMA,
    ],
)
def _sc_edge(featflat, elrflat, srcs, dsts, wgts, zrows, rst, exh,
             fb0, fb1, src2, dst2, wb, exb1, didx, exc2, crow2, iv0, iv1,
             srowb, drowb, dbuf, acc, den,
             st0, st1, g0s, g1s, i0s, i1s, ssem):
    cid = lax.axis_index("c")
    sid = lax.axis_index("s")
    iota = lax.iota(I32, 16)
    rot_idx = (iota + 8) % 16
    lmask8 = iota < 8
    iota_cc = iota * CC
    zeros16 = jnp.zeros((16,), F32)
    NG = CC // 16
    NTAIL = NCH - NFULL * NT
    fbs = (fb0, fb1)
    gsems = (g0s, g1s)
    isems = (i0s, i1s)
    stsems = (st0, st1)
    nch_t = jnp.where(sid < NTAIL, NFULL + 1, NFULL)

    @pl.loop(0, DEN_SL // 16)
    def _(r):
        dbuf[pl.ds(r * 16, 16)] = zeros16

    pltpu.sync_copy(dbuf, den.at[pl.ds(sid * DEN_SL, DEN_SL)])
    plsc.subcore_barrier()

    def do_chunk_a(k):
        c = sid + NT * k
        ebase = cid * E + c * CC
        d1 = pltpu.async_copy(srcs.at[pl.ds(ebase, CC)], src2.at[0], st0)
        d2 = pltpu.async_copy(dsts.at[pl.ds(ebase, CC)], dst2.at[0], st0)
        d3 = pltpu.async_copy(wgts.at[pl.ds(ebase, CC)], wb, st0)
        d1.wait()
        d2.wait()
        d3.wait()
        for q in range(NG):
            ds16 = pl.ds(q * 16, 16)
            srowb[ds16] = src2[0, ds16] + cid * NP
            drowb[ds16] = dst2[0, ds16] + cid * NP
        e1 = pltpu.async_copy(elrflat.at[srowb], fb0, g0s)
        e2 = pltpu.async_copy(elrflat.at[drowb], fb1, g1s)
        e1.wait()
        e2.wait()

        @pl.loop(0, NG)
        def _(q):
            q16 = q * 16
            wv = wb[pl.ds(q16, 16)]
            dv = dst2[0, pl.ds(q16, 16)]
            for h in range(H):
                didx[h, pl.ds(q16, 16)] = dv + h * NP
            for j in range(16):
                er_ = q16 + j
                ev = fb0[er_, pl.ds(0, 16)]
                rv = fb1[er_, pl.ds(0, 16)]
                s = (ev + jnp.take(rv, rot_idx)) * wv[j]
                s = jnp.where(s > 0, s, 0.2 * s)
                plsc.store_scatter(exb1, [iota_cc + er_], jnp.exp(s),
                                   mask=lmask8)

        cb = (cid * NCH + c) * (H * CC)
        wx = pltpu.async_copy(exb1, exh.at[pl.ds(cb, H * CC)], st1)
        waits = [pltpu.async_copy(exb1.at[pl.ds(h * CC, CC)],
                                  den.at[didx.at[h]], ssem, add=True)
                 for h in range(H)]
        wx.wait()
        for wd in waits:
            wd.wait()

    @pl.loop(0, NFULL)
    def _(k):
        do_chunk_a(k)

    @pl.when(sid < NTAIL)
    def _():
        do_chunk_a(NFULL)

    plsc.subcore_barrier()

    d0 = sid * DEN_SL
    pltpu.sync_copy(den.at[pl.ds(d0, DEN_SL)], dbuf)

    @pl.loop(0, DEN_SL // 16)
    def _(r):
        ds16 = pl.ds(r * 16, 16)
        dbuf[ds16] = 1.0 / (dbuf[ds16] + 1e-9)

    pltpu.sync_copy(dbuf, den.at[pl.ds(d0, DEN_SL)])
    plsc.subcore_barrier()

    @pl.loop(0, 4)
    def _(g):
        h0 = 2 * g
        pltpu.sync_copy(zrows, acc.at[pl.ds(sid * NSL, NSL), :])
        plsc.subcore_barrier()
        goff = (cid * 4 + g) * NP

        def stage_c(k, b):
            c = sid + NT * k
            ebase = cid * E + c * CC
            cb = (cid * NCH + c) * (H * CC)
            return [
                pltpu.async_copy(srcs.at[pl.ds(ebase, CC)], src2.at[b],
                                 stsems[b]),
                pltpu.async_copy(dsts.at[pl.ds(ebase, CC)], dst2.at[b],
                                 stsems[b]),
                pltpu.async_copy(exh.at[pl.ds(cb + h0 * CC, 2 * CC)],
                                 exc2.at[b], stsems[b]),
            ]

        def wait_stage_c(k, b):
            c = sid + NT * k
            ebase = cid * E + c * CC
            cb = (cid * NCH + c) * (H * CC)
            pltpu.make_async_copy(srcs.at[pl.ds(ebase, CC)], src2.at[b],
                                  stsems[b]).wait()
            pltpu.make_async_copy(dsts.at[pl.ds(ebase, CC)], dst2.at[b],
                                  stsems[b]).wait()
            pltpu.make_async_copy(exh.at[pl.ds(cb + h0 * CC, 2 * CC)],
                                  exc2.at[b], stsems[b]).wait()

        def fire_gathers_c(b):
            for q in range(NG):
                ds16 = pl.ds(q * 16, 16)
                dv = dst2[b, ds16]
                crow2[b, ds16] = src2[b, ds16] + goff
                didx[2 * b, ds16] = dv + h0 * NP
                didx[2 * b + 1, ds16] = dv + (h0 + 1) * NP
            pltpu.async_copy(featflat.at[crow2.at[b]], fbs[b], gsems[b])
            pltpu.async_copy(den.at[didx.at[2 * b]], iv0.at[b], isems[b])
            pltpu.async_copy(den.at[didx.at[2 * b + 1]], iv1.at[b], isems[b])

        def consume_c(b):
            fb = fbs[b]
            pltpu.make_async_copy(featflat.at[crow2.at[b]], fb,
                                  gsems[b]).wait()
            pltpu.make_async_copy(den.at[didx.at[2 * b]], iv0.at[b],
                                  isems[b]).wait()
            pltpu.make_async_copy(den.at[didx.at[2 * b + 1]], iv1.at[b],
                                  isems[b]).wait()

            @plsc.parallel_loop(0, NG, unroll=2)
            def _(q):
                q16 = q * 16
                ds16 = pl.ds(q16, 16)
                a0 = exc2[b, ds16] * iv0[b, ds16]
                a1 = exc2[b, pl.ds(CC + q16, 16)] * iv1[b, ds16]
                for j in range(16):
                    er_ = q16 + j
                    s0 = a0[j]
                    s1 = a1[j]
                    for kk in range(4):
                        dsk = pl.ds(kk * 16, 16)
                        fb[er_, dsk] = fb[er_, dsk] * s0
                    for kk in range(4, 8):
                        dsk = pl.ds(kk * 16, 16)
                        fb[er_, dsk] = fb[er_, dsk] * s1

            pltpu.sync_copy(fb, acc.at[dst2.at[b]], add=True)

        for d in stage_c(0, 0):
            d.wait()
        fire_gathers_c(0)

        @pl.loop(0, NFULL // 2)
        def _(m):
            k0 = 2 * m
            stage_c(k0 + 1, 1)
            consume_c(0)
            wait_stage_c(k0 + 1, 1)
            fire_gathers_c(1)
            knext = k0 + 2

            @pl.when(knext < nch_t)
            def _():
                stage_c(knext, 0)
            consume_c(1)

            @pl.when(knext < nch_t)
            def _():
                wait_stage_c(knext, 0)
                fire_gathers_c(0)

        @pl.when(sid < NTAIL)
        def _():
            consume_c(0)

        plsc.subcore_barrier()
        r0 = sid * NSL
        pltpu.sync_copy(acc.at[pl.ds(r0, NSL), :],
                        rst.at[cid, g, pl.ds(r0, NSL), :])
        plsc.subcore_barrier()


def _elu_z(rst_ref, bs_ref, p):
    z = jnp.concatenate([rst_ref[p, g] for g in range(4)], axis=1)
    z = z + bs_ref[p][None, :]
    return jnp.where(z > 0, z, jnp.exp(jnp.minimum(z, 0.0)) - 1.0)


def _sem_partial_body(rst_ref, bs_ref, w1_ref, b1_ref, w2_ref, o_ref):
    outs = []
    for p in range(2):
        z = _elu_z(rst_ref, bs_ref, p)
        t = jnp.tanh(jnp.dot(z, w1_ref[...], preferred_element_type=F32)
                     + b1_ref[0][None, :])
        s = jnp.sum(t * w2_ref[:, 0][None, :])
        outs.append(s.reshape(1, 1, 1))
    o_ref[...] = jnp.concatenate(outs, axis=2)


def _sem_partial(rst, bs, sW1, sb1, sW2):
    return pl.pallas_call(
        _sem_partial_body,
        grid=(NB2,),
        in_specs=[
            pl.BlockSpec((2, 4, BN2, 128), lambda i: (0, 0, i, 0)),
            pl.BlockSpec((2, HD), lambda i: (0, 0)),
            pl.BlockSpec((HD, SEM_HID), lambda i: (0, 0)),
            pl.BlockSpec((1, SEM_HID), lambda i: (0, 0)),
            pl.BlockSpec((SEM_HID, 1), lambda i: (0, 0)),
        ],
        out_specs=pl.BlockSpec((1, 1, 2), lambda i: (i, 0, 0)),
        out_shape=jax.ShapeDtypeStruct((NB2, 1, 2), F32),
    )(rst, bs, sW1, sb1, sW2)


def _combine_body(rst_ref, bs_ref, p_ref, o_ref):
    s = jnp.sum(p_ref[...], axis=0) / N
    s = s - jnp.max(s)
    bexp = jnp.exp(s)
    beta = bexp / jnp.sum(bexp)
    z0 = _elu_z(rst_ref, bs_ref, 0)
    z1 = _elu_z(rst_ref, bs_ref, 1)
    o_ref[...] = z0 * beta[0] + z1 * beta[1]


def _combine(rst, bs, partials):
    return pl.pallas_call(
        _combine_body,
        grid=(NB2,),
        in_specs=[
            pl.BlockSpec((2, 4, BN2, 128), lambda i: (0, 0, i, 0)),
            pl.BlockSpec((2, HD), lambda i: (0, 0)),
            pl.BlockSpec((NB2, 2), lambda i: (0, 0)),
        ],
        out_specs=pl.BlockSpec((BN2, HD), lambda i: (i, 0)),
        out_shape=jax.ShapeDtypeStruct((N, HD), F32),
    )(rst, bs, partials)


def kernel(x, edge_index_0, edge_weight_0, edge_index_1, edge_weight_1,
           W_0, attn_l_0, attn_r_0, bias_0,
           W_1, attn_l_1, attn_r_1, bias_1,
           sem_W1, sem_b1, sem_W2):
    Ws = jnp.stack([W_0, W_1])
    als = jnp.stack([attn_l_0, attn_l_1])
    ars = jnp.stack([attn_r_0, attn_r_1])
    xp = jnp.pad(x, ((0, NP - N), (0, 0)))
    featg, elr = _prep(xp, Ws, als, ars)
    featflat = featg.reshape(8 * NP, 128)
    elrflat = elr.reshape(2 * NP, 128)
    srcs = jnp.concatenate([edge_index_0[0], edge_index_1[0]]).astype(I32)
    dsts = jnp.concatenate([edge_index_0[1], edge_index_1[1]]).astype(I32)
    wgts = jnp.concatenate([edge_weight_0, edge_weight_1])
    zrows = jnp.zeros((NSL, 128), F32)
    rst, _ex = _sc_edge(featflat, elrflat, srcs, dsts, wgts, zrows)
    bs = jnp.stack([bias_0, bias_1])
    partials = _sem_partial(rst, bs, sem_W1, sem_b1.reshape(1, SEM_HID), sem_W2)
    return _combine(rst, bs, partials.reshape(NB2, 2))


# speedup vs baseline: 35.8203x; 1.2119x over previous
import functools

import jax
import jax.numpy as jnp
from jax import lax
from jax.experimental import pallas as pl
from jax.experimental.pallas import tpu as pltpu
from jax.experimental.pallas import tpu_sc as plsc

N = 10000
NP = 10240
E = 320000
D_IN = 128
H = 8
D_OUT = 64
HD = H * D_OUT
SEM_HID = 128

BN = 512
NBP = NP // BN
BN2 = 400
NB2 = N // BN2
NT = 16
CC = 128
NCH = E // CC
NFULL = NCH // NT
NSL = NP // NT
DEN_SL = H * NP // NT

F32 = jnp.float32
I32 = jnp.int32


def _prep_body(x_ref, w_ref, al_ref, ar_ref, feat_ref, elr_ref):
    f = jnp.dot(x_ref[...], w_ref[0], preferred_element_type=F32)
    for k in range(4):
        feat_ref[k] = f[:, 128 * k:128 * (k + 1)]
    cols = []
    for h in range(H):
        fh = f[:, D_OUT * h:D_OUT * (h + 1)]
        cols.append(jnp.sum(fh * al_ref[0, h][None, :], axis=1, keepdims=True))
    for h in range(H):
        fh = f[:, D_OUT * h:D_OUT * (h + 1)]
        cols.append(jnp.sum(fh * ar_ref[0, h][None, :], axis=1, keepdims=True))
    cols.append(jnp.zeros((BN, 128 - 2 * H), F32))
    elr_ref[0] = jnp.concatenate(cols, axis=1)


def _prep(xp, Ws, als, ars):
    return pl.pallas_call(
        _prep_body,
        grid=(NBP, 2),
        in_specs=[
            pl.BlockSpec((BN, D_IN), lambda i, p: (i, 0)),
            pl.BlockSpec((1, D_IN, HD), lambda i, p: (p, 0, 0)),
            pl.BlockSpec((1, H, D_OUT), lambda i, p: (p, 0, 0)),
            pl.BlockSpec((1, H, D_OUT), lambda i, p: (p, 0, 0)),
        ],
        out_specs=[
            pl.BlockSpec((4, BN, 128), lambda i, p: (p, i, 0)),
            pl.BlockSpec((1, BN, 128), lambda i, p: (p, i, 0)),
        ],
        out_shape=[
            jax.ShapeDtypeStruct((8, NP, 128), F32),
            jax.ShapeDtypeStruct((2, NP, 128), F32),
        ],
    )(xp, Ws, als, ars)


_mesh = plsc.VectorSubcoreMesh(core_axis_name="c", subcore_axis_name="s")


@functools.partial(
    pl.kernel,
    out_type=[
        jax.ShapeDtypeStruct((2, 4, NP, 128), F32),
        jax.ShapeDtypeStruct((2 * H * E,), F32),
    ],
    mesh=_mesh,
    compiler_params=pltpu.CompilerParams(needs_layout_passes=False),
    scratch_types=[
        pltpu.VMEM((CC, 128), F32),
        pltpu.VMEM((CC, 128), F32),
        pltpu.VMEM((2, CC), I32),
        pltpu.VMEM((2, CC), I32),
        pltpu.VMEM((CC,), F32),
        pltpu.VMEM((H * CC,), F32),
        pltpu.VMEM((H, CC), I32),
        pltpu.VMEM((2, 2 * CC), F32),
        pltpu.VMEM((2, CC), I32),
        pltpu.VMEM((2, CC), F32),
        pltpu.VMEM((2, CC), F32),
        pltpu.VMEM((CC,), I32),
        pltpu.VMEM((CC,), I32),
        pltpu.VMEM((DEN_SL,), F32),
        pltpu.VMEM_SHARED((NP, 128), F32),
        pltpu.VMEM_SHARED((H * NP,), F32),
        pltpu.SemaphoreType.DMA,
        pltpu.SemaphoreType.DMA,
        pltpu.SemaphoreType.DMA,
        pltpu.SemaphoreType.DMA,
        pltpu.SemaphoreType.DMA,
        pltpu.SemaphoreType.DMA,
        pltpu.SemaphoreType.DMA,
    ],
)
def _sc_edge(featflat, elrflat, srcs, dsts, wgts, zrows, rst, exh,
             fb0, fb1, src2, dst2, wb, exb1, didx, exc2, crow2, iv0, iv1,
             srowb, drowb, dbuf, acc, den,
             st0, st1, g0s, g1s, i0s, i1s, ssem):
    cid = lax.axis_index("c")
    sid = lax.axis_index("s")
    iota = lax.iota(I32, 16)
    rot_idx = (iota + 8) % 16
    lmask8 = iota < 8
    iota_cc = iota * CC
    zeros16 = jnp.zeros((16,), F32)
    NG = CC // 16
    NTAIL = NCH - NFULL * NT
    fbs = (fb0, fb1)
    gsems = (g0s, g1s)
    isems = (i0s, i1s)
    stsems = (st0, st1)
    nch_t = jnp.where(sid < NTAIL, NFULL + 1, NFULL)

    @pl.loop(0, DEN_SL // 16)
    def _(r):
        dbuf[pl.ds(r * 16, 16)] = zeros16

    pltpu.sync_copy(dbuf, den.at[pl.ds(sid * DEN_SL, DEN_SL)])
    plsc.subcore_barrier()

    def do_chunk_a(k):
        c = sid + NT * k
        ebase = cid * E + c * CC
        d1 = pltpu.async_copy(srcs.at[pl.ds(ebase, CC)], src2.at[0], st0)
        d2 = pltpu.async_copy(dsts.at[pl.ds(ebase, CC)], dst2.at[0], st0)
        d3 = pltpu.async_copy(wgts.at[pl.ds(ebase, CC)], wb, st0)
        d1.wait()
        d2.wait()
        d3.wait()
        for q in range(NG):
            ds16 = pl.ds(q * 16, 16)
            srowb[ds16] = src2[0, ds16] + cid * NP
            drowb[ds16] = dst2[0, ds16] + cid * NP
        e1 = pltpu.async_copy(elrflat.at[srowb], fb0, g0s)
        e2 = pltpu.async_copy(elrflat.at[drowb], fb1, g1s)
        e1.wait()
        e2.wait()

        @pl.loop(0, NG)
        def _(q):
            q16 = q * 16
            wv = wb[pl.ds(q16, 16)]
            dv = dst2[0, pl.ds(q16, 16)]
            for h in range(H):
                didx[h, pl.ds(q16, 16)] = dv + h * NP
            for j in range(16):
                er_ = q16 + j
                ev = fb0[er_, pl.ds(0, 16)]
                rv = fb1[er_, pl.ds(0, 16)]
                s = (ev + jnp.take(rv, rot_idx)) * wv[j]
                s = jnp.where(s > 0, s, 0.2 * s)
                plsc.store_scatter(exb1, [iota_cc + er_], jnp.exp(s),
                                   mask=lmask8)

        cb = (cid * NCH + c) * (H * CC)
        wx = pltpu.async_copy(exb1, exh.at[pl.ds(cb, H * CC)], st1)
        waits = [pltpu.async_copy(exb1.at[pl.ds(h * CC, CC)],
                                  den.at[didx.at[h]], ssem, add=True)
                 for h in range(H)]
        wx.wait()
        for wd in waits:
            wd.wait()

    @pl.loop(0, NFULL)
    def _(k):
        do_chunk_a(k)

    @pl.when(sid < NTAIL)
    def _():
        do_chunk_a(NFULL)

    plsc.subcore_barrier()

    d0 = sid * DEN_SL
    pltpu.sync_copy(den.at[pl.ds(d0, DEN_SL)], dbuf)

    @pl.loop(0, DEN_SL // 16)
    def _(r):
        ds16 = pl.ds(r * 16, 16)
        dbuf[ds16] = 1.0 / (dbuf[ds16] + 1e-9)

    pltpu.sync_copy(dbuf, den.at[pl.ds(d0, DEN_SL)])
    plsc.subcore_barrier()

    @pl.loop(0, 4)
    def _(g):
        h0 = 2 * g
        pltpu.sync_copy(zrows, acc.at[pl.ds(sid * NSL, NSL), :])
        plsc.subcore_barrier()
        goff = (cid * 4 + g) * NP

        def stage_c(k, b):
            c = sid + NT * k
            ebase = cid * E + c * CC
            cb = (cid * NCH + c) * (H * CC)
            return [
                pltpu.async_copy(srcs.at[pl.ds(ebase, CC)], src2.at[b],
                                 stsems[b]),
                pltpu.async_copy(dsts.at[pl.ds(ebase, CC)], dst2.at[b],
                                 stsems[b]),
                pltpu.async_copy(exh.at[pl.ds(cb + h0 * CC, 2 * CC)],
                                 exc2.at[b], stsems[b]),
            ]

        def wait_stage_c(k, b):
            c = sid + NT * k
            ebase = cid * E + c * CC
            cb = (cid * NCH + c) * (H * CC)
            pltpu.make_async_copy(srcs.at[pl.ds(ebase, CC)], src2.at[b],
                                  stsems[b]).wait()
            pltpu.make_async_copy(dsts.at[pl.ds(ebase, CC)], dst2.at[b],
                                  stsems[b]).wait()
            pltpu.make_async_copy(exh.at[pl.ds(cb + h0 * CC, 2 * CC)],
                                  exc2.at[b], stsems[b]).wait()

        def fire_gathers_c(b):
            for q in range(NG):
                ds16 = pl.ds(q * 16, 16)
                dv = dst2[b, ds16]
                crow2[b, ds16] = src2[b, ds16] + goff
                didx[2 * b, ds16] = dv + h0 * NP
                didx[2 * b + 1, ds16] = dv + (h0 + 1) * NP
            pltpu.async_copy(featflat.at[crow2.at[b]], fbs[b], gsems[b])
            pltpu.async_copy(den.at[didx.at[2 * b]], iv0.at[b], isems[b])
            pltpu.async_copy(den.at[didx.at[2 * b + 1]], iv1.at[b], isems[b])

        def consume_c(b):
            fb = fbs[b]
            pltpu.make_async_copy(featflat.at[crow2.at[b]], fb,
                                  gsems[b]).wait()
            pltpu.make_async_copy(den.at[didx.at[2 * b]], iv0.at[b],
                                  isems[b]).wait()
            pltpu.make_async_copy(den.at[didx.at[2 * b + 1]], iv1.at[b],
                                  isems[b]).wait()

            @plsc.parallel_loop(0, NG, unroll=2)
            def _(q):
                q16 = q * 16
                ds16 = pl.ds(q16, 16)
                a0 = exc2[b, ds16] * iv0[b, ds16]
                a1 = exc2[b, pl.ds(CC + q16, 16)] * iv1[b, ds16]
                for j in range(16):
                    er_ = q16 + j
                    s0 = a0[j]
                    s1 = a1[j]
                    for kk in range(4):
                        dsk = pl.ds(kk * 16, 16)
                        fb[er_, dsk] = fb[er_, dsk] * s0
                    for kk in range(4, 8):
                        dsk = pl.ds(kk * 16, 16)
                        fb[er_, dsk] = fb[er_, dsk] * s1

            pltpu.sync_copy(fb, acc.at[dst2.at[b]], add=True)

        for d in stage_c(0, 0):
            d.wait()
        fire_gathers_c(0)
        stage_c(1, 1)

        @pl.loop(0, NFULL // 2)
        def _(m):
            wait_stage_c(2 * m + 1, 1)
            fire_gathers_c(1)
            consume_c(0)

            @pl.when(2 * m + 2 < nch_t)
            def _():
                stage_c(2 * m + 2, 0)
            @pl.when(2 * m + 2 < nch_t)
            def _():
                wait_stage_c(2 * m + 2, 0)
                fire_gathers_c(0)
            consume_c(1)

            @pl.when(2 * m + 3 < nch_t)
            def _():
                stage_c(2 * m + 3, 1)

        @pl.when(sid < NTAIL)
        def _():
            consume_c(0)

        plsc.subcore_barrier()
        r0 = sid * NSL
        pltpu.sync_copy(acc.at[pl.ds(r0, NSL), :],
                        rst.at[cid, g, pl.ds(r0, NSL), :])
        plsc.subcore_barrier()


def _elu_z(rst_ref, bs_ref, p):
    z = jnp.concatenate([rst_ref[p, g] for g in range(4)], axis=1)
    z = z + bs_ref[p][None, :]
    return jnp.where(z > 0, z, jnp.exp(jnp.minimum(z, 0.0)) - 1.0)


def _sem_partial_body(rst_ref, bs_ref, w1_ref, b1_ref, w2_ref, o_ref):
    outs = []
    for p in range(2):
        z = _elu_z(rst_ref, bs_ref, p)
        t = jnp.tanh(jnp.dot(z, w1_ref[...], preferred_element_type=F32)
                     + b1_ref[0][None, :])
        s = jnp.sum(t * w2_ref[:, 0][None, :])
        outs.append(s.reshape(1, 1, 1))
    o_ref[...] = jnp.concatenate(outs, axis=2)


def _sem_partial(rst, bs, sW1, sb1, sW2):
    return pl.pallas_call(
        _sem_partial_body,
        grid=(NB2,),
        in_specs=[
            pl.BlockSpec((2, 4, BN2, 128), lambda i: (0, 0, i, 0)),
            pl.BlockSpec((2, HD), lambda i: (0, 0)),
            pl.BlockSpec((HD, SEM_HID), lambda i: (0, 0)),
            pl.BlockSpec((1, SEM_HID), lambda i: (0, 0)),
            pl.BlockSpec((SEM_HID, 1), lambda i: (0, 0)),
        ],
        out_specs=pl.BlockSpec((1, 1, 2), lambda i: (i, 0, 0)),
        out_shape=jax.ShapeDtypeStruct((NB2, 1, 2), F32),
    )(rst, bs, sW1, sb1, sW2)


def _combine_body(rst_ref, bs_ref, p_ref, o_ref):
    s = jnp.sum(p_ref[...], axis=0) / N
    s = s - jnp.max(s)
    bexp = jnp.exp(s)
    beta = bexp / jnp.sum(bexp)
    z0 = _elu_z(rst_ref, bs_ref, 0)
    z1 = _elu_z(rst_ref, bs_ref, 1)
    o_ref[...] = z0 * beta[0] + z1 * beta[1]


def _combine(rst, bs, partials):
    return pl.pallas_call(
        _combine_body,
        grid=(NB2,),
        in_specs=[
            pl.BlockSpec((2, 4, BN2, 128), lambda i: (0, 0, i, 0)),
            pl.BlockSpec((2, HD), lambda i: (0, 0)),
            pl.BlockSpec((NB2, 2), lambda i: (0, 0)),
        ],
        out_specs=pl.BlockSpec((BN2, HD), lambda i: (i, 0)),
        out_shape=jax.ShapeDtypeStruct((N, HD), F32),
    )(rst, bs, partials)


def kernel(x, edge_index_0, edge_weight_0, edge_index_1, edge_weight_1,
           W_0, attn_l_0, attn_r_0, bias_0,
           W_1, attn_l_1, attn_r_1, bias_1,
           sem_W1, sem_b1, sem_W2):
    Ws = jnp.stack([W_0, W_1])
    als = jnp.stack([attn_l_0, attn_l_1])
    ars = jnp.stack([attn_r_0, attn_r_1])
    xp = jnp.pad(x, ((0, NP - N), (0, 0)))
    featg, elr = _prep(xp, Ws, als, ars)
    featflat = featg.reshape(8 * NP, 128)
    elrflat = elr.reshape(2 * NP, 128)
    srcs = jnp.concatenate([edge_index_0[0], edge_index_1[0]]).astype(I32)
    dsts = jnp.concatenate([edge_index_0[1], edge_index_1[1]]).astype(I32)
    wgts = jnp.concatenate([edge_weight_0, edge_weight_1])
    zrows = jnp.zeros((NSL, 128), F32)
    rst, _ex = _sc_edge(featflat, elrflat, srcs, dsts, wgts, zrows)
    bs = jnp.stack([bias_0, bias_1])
    partials = _sem_partial(rst, bs, sem_W1, sem_b1.reshape(1, SEM_HID), sem_W2)
    return _combine(rst, bs, partials.reshape(NB2, 2))


# speedup vs baseline: 35.9047x; 1.0024x over previous
import functools

import jax
import jax.numpy as jnp
from jax import lax
from jax.experimental import pallas as pl
from jax.experimental.pallas import tpu as pltpu
from jax.experimental.pallas import tpu_sc as plsc

N = 10000
NP = 10240
E = 320000
D_IN = 128
H = 8
D_OUT = 64
HD = H * D_OUT
SEM_HID = 128

BN = 512
NBP = NP // BN
BN2 = 400
NB2 = N // BN2
NT = 16
CC = 128
NCH = E // CC
NFULL = NCH // NT
NSL = NP // NT
DEN_SL = H * NP // NT

F32 = jnp.float32
I32 = jnp.int32


def _prep_body(x_ref, w_ref, al_ref, ar_ref, feat_ref, elr_ref):
    f = jnp.dot(x_ref[...], w_ref[0], preferred_element_type=F32)
    for k in range(4):
        feat_ref[k] = f[:, 128 * k:128 * (k + 1)]
    cols = []
    for h in range(H):
        fh = f[:, D_OUT * h:D_OUT * (h + 1)]
        cols.append(jnp.sum(fh * al_ref[0, h][None, :], axis=1, keepdims=True))
    for h in range(H):
        fh = f[:, D_OUT * h:D_OUT * (h + 1)]
        cols.append(jnp.sum(fh * ar_ref[0, h][None, :], axis=1, keepdims=True))
    cols.append(jnp.zeros((BN, 128 - 2 * H), F32))
    elr_ref[0] = jnp.concatenate(cols, axis=1)


def _prep(xp, Ws, als, ars):
    return pl.pallas_call(
        _prep_body,
        grid=(NBP, 2),
        in_specs=[
            pl.BlockSpec((BN, D_IN), lambda i, p: (i, 0)),
            pl.BlockSpec((1, D_IN, HD), lambda i, p: (p, 0, 0)),
            pl.BlockSpec((1, H, D_OUT), lambda i, p: (p, 0, 0)),
            pl.BlockSpec((1, H, D_OUT), lambda i, p: (p, 0, 0)),
        ],
        out_specs=[
            pl.BlockSpec((4, BN, 128), lambda i, p: (p, i, 0)),
            pl.BlockSpec((1, BN, 128), lambda i, p: (p, i, 0)),
        ],
        out_shape=[
            jax.ShapeDtypeStruct((8, NP, 128), F32),
            jax.ShapeDtypeStruct((2, NP, 128), F32),
        ],
    )(xp, Ws, als, ars)


_mesh = plsc.VectorSubcoreMesh(core_axis_name="c", subcore_axis_name="s")


@functools.partial(
    pl.kernel,
    out_type=[
        jax.ShapeDtypeStruct((2, 4, NP, 128), F32),
        jax.ShapeDtypeStruct((2 * H * E,), F32),
    ],
    mesh=_mesh,
    compiler_params=pltpu.CompilerParams(needs_layout_passes=False),
    scratch_types=[
        pltpu.VMEM((CC, 128), F32),
        pltpu.VMEM((CC, 128), F32),
        pltpu.VMEM((2, CC), I32),
        pltpu.VMEM((2, CC), I32),
        pltpu.VMEM((CC,), F32),
        pltpu.VMEM((H * CC,), F32),
        pltpu.VMEM((H, CC), I32),
        pltpu.VMEM((2, 2 * CC), F32),
        pltpu.VMEM((2, CC), I32),
        pltpu.VMEM((2, CC), F32),
        pltpu.VMEM((2, CC), F32),
        pltpu.VMEM((CC,), I32),
        pltpu.VMEM((CC,), I32),
        pltpu.VMEM((DEN_SL,), F32),
        pltpu.VMEM_SHARED((NP, 128), F32),
        pltpu.VMEM_SHARED((H * NP,), F32),
        pltpu.SemaphoreType.DMA,
        pltpu.SemaphoreType.DMA,
        pltpu.SemaphoreType.DMA,
        pltpu.SemaphoreType.DMA,
        pltpu.SemaphoreType.DMA,
        pltpu.SemaphoreType.DMA,
        pltpu.SemaphoreType.DMA,
    ],
)
def _sc_edge(featflat, elrflat, srcs, dsts, wgts, zrows, rst, exh,
             fb0, fb1, src2, dst2, wb, exb1, didx, exc2, crow2, iv0, iv1,
             srowb, drowb, dbuf, acc, den,
             st0, st1, g0s, g1s, i0s, i1s, ssem):
    cid = lax.axis_index("c")
    sid = lax.axis_index("s")
    iota = lax.iota(I32, 16)
    rot_idx = (iota + 8) % 16
    lmask8 = iota < 8
    iota_cc = iota * CC
    zeros16 = jnp.zeros((16,), F32)
    NG = CC // 16
    NTAIL = NCH - NFULL * NT
    fbs = (fb0, fb1)
    gsems = (g0s, g1s)
    isems = (i0s, i1s)
    stsems = (st0, st1)
    nch_t = jnp.where(sid < NTAIL, NFULL + 1, NFULL)

    @pl.loop(0, DEN_SL // 16)
    def _(r):
        dbuf[pl.ds(r * 16, 16)] = zeros16

    pltpu.sync_copy(dbuf, den.at[pl.ds(sid * DEN_SL, DEN_SL)])
    plsc.subcore_barrier()

    def do_chunk_a(k):
        c = sid + NT * k
        ebase = cid * E + c * CC
        d1 = pltpu.async_copy(srcs.at[pl.ds(ebase, CC)], src2.at[0], st0)
        d2 = pltpu.async_copy(dsts.at[pl.ds(ebase, CC)], dst2.at[0], st0)
        d3 = pltpu.async_copy(wgts.at[pl.ds(ebase, CC)], wb, st0)
        d1.wait()
        d2.wait()
        d3.wait()
        for q in range(NG):
            ds16 = pl.ds(q * 16, 16)
            srowb[ds16] = src2[0, ds16] + cid * NP
            drowb[ds16] = dst2[0, ds16] + cid * NP
        e1 = pltpu.async_copy(elrflat.at[srowb], fb0, g0s)
        e2 = pltpu.async_copy(elrflat.at[drowb], fb1, g1s)
        e1.wait()
        e2.wait()

        @pl.loop(0, NG)
        def _(q):
            q16 = q * 16
            wv = wb[pl.ds(q16, 16)]
            dv = dst2[0, pl.ds(q16, 16)]
            for h in range(H):
                didx[h, pl.ds(q16, 16)] = dv + h * NP
            for j in range(16):
                er_ = q16 + j
                ev = fb0[er_, pl.ds(0, 16)]
                rv = fb1[er_, pl.ds(0, 16)]
                s = (ev + jnp.take(rv, rot_idx)) * wv[j]
                s = jnp.where(s > 0, s, 0.2 * s)
                plsc.store_scatter(exb1, [iota_cc + er_], jnp.exp(s),
                                   mask=lmask8)

        cb = (cid * NCH + c) * (H * CC)
        wx = pltpu.async_copy(exb1, exh.at[pl.ds(cb, H * CC)], st1)
        waits = [pltpu.async_copy(exb1.at[pl.ds(h * CC, CC)],
                                  den.at[didx.at[h]], ssem, add=True)
                 for h in range(H)]
        wx.wait()
        for wd in waits:
            wd.wait()

    @pl.loop(0, NFULL)
    def _(k):
        do_chunk_a(k)

    @pl.when(sid < NTAIL)
    def _():
        do_chunk_a(NFULL)

    plsc.subcore_barrier()

    d0 = sid * DEN_SL
    pltpu.sync_copy(den.at[pl.ds(d0, DEN_SL)], dbuf)

    @pl.loop(0, DEN_SL // 16)
    def _(r):
        ds16 = pl.ds(r * 16, 16)
        dbuf[ds16] = 1.0 / (dbuf[ds16] + 1e-9)

    pltpu.sync_copy(dbuf, den.at[pl.ds(d0, DEN_SL)])
    plsc.subcore_barrier()

    @pl.loop(0, 4)
    def _(g):
        h0 = 2 * g
        pltpu.sync_copy(zrows, acc.at[pl.ds(sid * NSL, NSL), :])
        plsc.subcore_barrier()
        goff = (cid * 4 + g) * NP

        def stage_c(k, b):
            c = sid + NT * k
            ebase = cid * E + c * CC
            cb = (cid * NCH + c) * (H * CC)
            return [
                pltpu.async_copy(srcs.at[pl.ds(ebase, CC)], src2.at[b],
                                 stsems[b]),
                pltpu.async_copy(dsts.at[pl.ds(ebase, CC)], dst2.at[b],
                                 stsems[b]),
                pltpu.async_copy(exh.at[pl.ds(cb + h0 * CC, 2 * CC)],
                                 exc2.at[b], stsems[b]),
            ]

        def wait_stage_c(k, b):
            c = sid + NT * k
            ebase = cid * E + c * CC
            cb = (cid * NCH + c) * (H * CC)
            pltpu.make_async_copy(srcs.at[pl.ds(ebase, CC)], src2.at[b],
                                  stsems[b]).wait()
            pltpu.make_async_copy(dsts.at[pl.ds(ebase, CC)], dst2.at[b],
                                  stsems[b]).wait()
            pltpu.make_async_copy(exh.at[pl.ds(cb + h0 * CC, 2 * CC)],
                                  exc2.at[b], stsems[b]).wait()

        def fire_gathers_c(b):
            for q in range(NG):
                ds16 = pl.ds(q * 16, 16)
                dv = dst2[b, ds16]
                crow2[b, ds16] = src2[b, ds16] + goff
                didx[2 * b, ds16] = dv + h0 * NP
                didx[2 * b + 1, ds16] = dv + (h0 + 1) * NP
            pltpu.async_copy(featflat.at[crow2.at[b]], fbs[b], gsems[b])
            pltpu.async_copy(den.at[didx.at[2 * b]], iv0.at[b], isems[b])
            pltpu.async_copy(den.at[didx.at[2 * b + 1]], iv1.at[b], isems[b])

        def consume_c(b):
            fb = fbs[b]
            pltpu.make_async_copy(featflat.at[crow2.at[b]], fb,
                                  gsems[b]).wait()
            pltpu.make_async_copy(den.at[didx.at[2 * b]], iv0.at[b],
                                  isems[b]).wait()
            pltpu.make_async_copy(den.at[didx.at[2 * b + 1]], iv1.at[b],
                                  isems[b]).wait()

            @plsc.parallel_loop(0, NG, unroll=4)
            def _(q):
                q16 = q * 16
                ds16 = pl.ds(q16, 16)
                a0 = exc2[b, ds16] * iv0[b, ds16]
                a1 = exc2[b, pl.ds(CC + q16, 16)] * iv1[b, ds16]
                for j in range(16):
                    er_ = q16 + j
                    s0 = a0[j]
                    s1 = a1[j]
                    for kk in range(4):
                        dsk = pl.ds(kk * 16, 16)
                        fb[er_, dsk] = fb[er_, dsk] * s0
                    for kk in range(4, 8):
                        dsk = pl.ds(kk * 16, 16)
                        fb[er_, dsk] = fb[er_, dsk] * s1

            pltpu.sync_copy(fb, acc.at[dst2.at[b]], add=True)

        for d in stage_c(0, 0):
            d.wait()
        fire_gathers_c(0)
        stage_c(1, 1)

        @pl.loop(0, NFULL // 2)
        def _(m):
            wait_stage_c(2 * m + 1, 1)
            fire_gathers_c(1)
            consume_c(0)

            @pl.when(2 * m + 2 < nch_t)
            def _():
                stage_c(2 * m + 2, 0)
            @pl.when(2 * m + 2 < nch_t)
            def _():
                wait_stage_c(2 * m + 2, 0)
                fire_gathers_c(0)
            consume_c(1)

            @pl.when(2 * m + 3 < nch_t)
            def _():
                stage_c(2 * m + 3, 1)

        @pl.when(sid < NTAIL)
        def _():
            consume_c(0)

        plsc.subcore_barrier()
        r0 = sid * NSL
        pltpu.sync_copy(acc.at[pl.ds(r0, NSL), :],
                        rst.at[cid, g, pl.ds(r0, NSL), :])
        plsc.subcore_barrier()


def _elu_z(rst_ref, bs_ref, p):
    z = jnp.concatenate([rst_ref[p, g] for g in range(4)], axis=1)
    z = z + bs_ref[p][None, :]
    return jnp.where(z > 0, z, jnp.exp(jnp.minimum(z, 0.0)) - 1.0)


def _sem_partial_body(rst_ref, bs_ref, w1_ref, b1_ref, w2_ref, o_ref):
    outs = []
    for p in range(2):
        z = _elu_z(rst_ref, bs_ref, p)
        t = jnp.tanh(jnp.dot(z, w1_ref[...], preferred_element_type=F32)
                     + b1_ref[0][None, :])
        s = jnp.sum(t * w2_ref[:, 0][None, :])
        outs.append(s.reshape(1, 1, 1))
    o_ref[...] = jnp.concatenate(outs, axis=2)


def _sem_partial(rst, bs, sW1, sb1, sW2):
    return pl.pallas_call(
        _sem_partial_body,
        grid=(NB2,),
        in_specs=[
            pl.BlockSpec((2, 4, BN2, 128), lambda i: (0, 0, i, 0)),
            pl.BlockSpec((2, HD), lambda i: (0, 0)),
            pl.BlockSpec((HD, SEM_HID), lambda i: (0, 0)),
            pl.BlockSpec((1, SEM_HID), lambda i: (0, 0)),
            pl.BlockSpec((SEM_HID, 1), lambda i: (0, 0)),
        ],
        out_specs=pl.BlockSpec((1, 1, 2), lambda i: (i, 0, 0)),
        out_shape=jax.ShapeDtypeStruct((NB2, 1, 2), F32),
    )(rst, bs, sW1, sb1, sW2)


def _combine_body(rst_ref, bs_ref, p_ref, o_ref):
    s = jnp.sum(p_ref[...], axis=0) / N
    s = s - jnp.max(s)
    bexp = jnp.exp(s)
    beta = bexp / jnp.sum(bexp)
    z0 = _elu_z(rst_ref, bs_ref, 0)
    z1 = _elu_z(rst_ref, bs_ref, 1)
    o_ref[...] = z0 * beta[0] + z1 * beta[1]


def _combine(rst, bs, partials):
    return pl.pallas_call(
        _combine_body,
        grid=(NB2,),
        in_specs=[
            pl.BlockSpec((2, 4, BN2, 128), lambda i: (0, 0, i, 0)),
            pl.BlockSpec((2, HD), lambda i: (0, 0)),
            pl.BlockSpec((NB2, 2), lambda i: (0, 0)),
        ],
        out_specs=pl.BlockSpec((BN2, HD), lambda i: (i, 0)),
        out_shape=jax.ShapeDtypeStruct((N, HD), F32),
    )(rst, bs, partials)


def kernel(x, edge_index_0, edge_weight_0, edge_index_1, edge_weight_1,
           W_0, attn_l_0, attn_r_0, bias_0,
           W_1, attn_l_1, attn_r_1, bias_1,
           sem_W1, sem_b1, sem_W2):
    Ws = jnp.stack([W_0, W_1])
    als = jnp.stack([attn_l_0, attn_l_1])
    ars = jnp.stack([attn_r_0, attn_r_1])
    xp = jnp.pad(x, ((0, NP - N), (0, 0)))
    featg, elr = _prep(xp, Ws, als, ars)
    featflat = featg.reshape(8 * NP, 128)
    elrflat = elr.reshape(2 * NP, 128)
    srcs = jnp.concatenate([edge_index_0[0], edge_index_1[0]]).astype(I32)
    dsts = jnp.concatenate([edge_index_0[1], edge_index_1[1]]).astype(I32)
    wgts = jnp.concatenate([edge_weight_0, edge_weight_1])
    zrows = jnp.zeros((NSL, 128), F32)
    rst, _ex = _sc_edge(featflat, elrflat, srcs, dsts, wgts, zrows)
    bs = jnp.stack([bias_0, bias_1])
    partials = _sem_partial(rst, bs, sem_W1, sem_b1.reshape(1, SEM_HID), sem_W2)
    return _combine(rst, bs, partials.reshape(NB2, 2))
